# baseline (speedup 1.0000x reference)
import functools

import jax
import jax.numpy as jnp
import numpy as np
from jax import lax
from jax.experimental import pallas as pl
from jax.experimental.pallas import tpu as pltpu
from jax.experimental.pallas import tpu_sc as plsc

N_EDGES_C = 2048
EDGE_DIM_C = 128
HIDDEN_C = 256
HEADS_C = 4
TOPK_C = 512
N_NODES_C = 1024
T_EDGES_C = 32768
SENT_C = N_EDGES_C * N_EDGES_C


def _dot(a, b):
    return lax.dot_general(a, b, (((1,), (0,)), ((), ())),
                           preferred_element_type=jnp.float32,
                           precision=lax.Precision.HIGHEST)


def _dot_t0(a, b):
    return lax.dot_general(a, b, (((0,), (0,)), ((), ())),
                           preferred_element_type=jnp.float32,
                           precision=lax.Precision.HIGHEST)


def _layernorm(x, g, b, eps=1e-5):
    mu = jnp.mean(x, -1, keepdims=True)
    var = jnp.mean((x - mu) ** 2, -1, keepdims=True)
    return (x - mu) / jnp.sqrt(var + eps) * g + b


_TABW = 128
_NW = 32
_ROWS_W = T_EDGES_C // _NW
_GCH = 128
_NCH = _ROWS_W // _GCH


def _gather_body(tab_hbm, gi_hbm, gj_hbm, tgi_hbm, tgj_hbm,
                 idx_v, rows_a, rows_b, sem_a, sem_b):
    wid = lax.axis_index("s") * 2 + lax.axis_index("c")
    base = wid * _ROWS_W
    for which in range(2):
        idx_hbm = (gi_hbm, gj_hbm)[which]
        out_hbm = (tgi_hbm, tgj_hbm)[which]
        pltpu.sync_copy(idx_hbm.at[pl.ds(wid * _NCH, _NCH)], idx_v)
        cps = []
        for c in range(_NCH):
            buf = (rows_a, rows_b)[c % 2]
            sem = (sem_a, sem_b)[c % 2]
            cp = pltpu.async_copy(tab_hbm.at[idx_v.at[c]], buf, sem)
            cps.append(cp)
            if c >= 1:
                cps[c - 1].wait()
                prev = (rows_a, rows_b)[(c - 1) % 2]
                pltpu.sync_copy(prev, out_hbm.at[pl.ds(base + (c - 1) * _GCH, _GCH)])
        cps[-1].wait()
        last = (rows_a, rows_b)[(_NCH - 1) % 2]
        pltpu.sync_copy(last, out_hbm.at[pl.ds(base + (_NCH - 1) * _GCH, _GCH)])


@jax.jit
def _gather(tab, gic, gjc):
    mesh = plsc.VectorSubcoreMesh(core_axis_name="c", subcore_axis_name="s")
    f = functools.partial(
        pl.kernel, mesh=mesh,
        out_type=(
            jax.ShapeDtypeStruct((T_EDGES_C, _TABW), jnp.float32),
            jax.ShapeDtypeStruct((T_EDGES_C, _TABW), jnp.float32),
        ),
        scratch_types=[
            pltpu.VMEM((_NCH, _GCH), jnp.int32),
            pltpu.VMEM((_GCH, _TABW), jnp.float32),
            pltpu.VMEM((_GCH, _TABW), jnp.float32),
            pltpu.SemaphoreType.DMA,
            pltpu.SemaphoreType.DMA,
        ],
    )(_gather_body)
    return f(tab, gic.reshape(T_EDGES_C // _GCH, _GCH), gjc.reshape(T_EDGES_C // _GCH, _GCH))


def _score_body(tgi_ref, tgj_ref, feats_ref,
                cW1_ref, cb1_ref, cW2_ref, cb2_ref,
                sW1_ref, sb1_ref, sW2_ref, sb2_ref,
                kW1_ref, kb1_ref, kW2_ref, kb2_ref,
                z_ref):
    dt = functools.partial(lax.dot_general, dimension_numbers=(((1,), (1,)), ((), ())),
                           preferred_element_type=jnp.float32)
    pair = jnp.concatenate([tgi_ref[...], tgj_ref[...]], axis=-1)
    c1 = jnp.maximum(dt(pair, cW1_ref[...]) + cb1_ref[...], 0.0)
    comp = jnp.maximum(dt(c1, cW2_ref[...]) + cb2_ref[...], 0.0)
    feats = feats_ref[...]
    s1 = jnp.maximum(dt(feats, sW1_ref[...]) + sb1_ref[...], 0.0)
    struct8 = dt(s1, sW2_ref[...]) + sb2_ref[...]
    allsc8 = jnp.concatenate([comp[:, :4], struct8[:, :4]], axis=-1)
    z1 = jnp.maximum(dt(allsc8, kW1_ref[...]) + kb1_ref[...], 0.0)
    z_ref[...] = dt(z1, kW2_ref[...]) + kb2_ref[...]


_SCORE_BLK = 4096


@jax.jit
def _score(tgi, tgj, feats8, p):
    z2 = lambda r, c: jnp.zeros((r, c), jnp.float32)
    cW2p = jnp.concatenate([p['cW2'], z2(4, HIDDEN_C)], 0)
    cb2p = jnp.concatenate([p['cb2'], jnp.zeros((4,), jnp.float32)])[None, :]
    sW1p = jnp.concatenate([p['sW1'], z2(64, 4)], 1)
    sW2p = jnp.concatenate([p['sW2'], z2(7, 64)], 0)
    sb2p = jnp.concatenate([p['sb2'], jnp.zeros((7,), jnp.float32)])[None, :]
    kW1p = jnp.concatenate([p['kW1'], z2(128, 3)], 1)
    kW2p = jnp.concatenate([p['kW2'], z2(7, 128)], 0)
    kb2p = jnp.concatenate([p['kb2'], jnp.zeros((7,), jnp.float32)])[None, :]
    ws = (p['cW1'], p['cb1'][None, :], cW2p, cb2p,
          sW1p, p['sb1'][None, :], sW2p, sb2p,
          kW1p, p['kb1'][None, :], kW2p, kb2p)
    grid = T_EDGES_C // _SCORE_BLK
    row_blk = lambda i: (i, 0)
    full = lambda i: (0, 0)
    in_specs = [
        pl.BlockSpec((_SCORE_BLK, tgi.shape[1]), row_blk),
        pl.BlockSpec((_SCORE_BLK, tgj.shape[1]), row_blk),
        pl.BlockSpec((_SCORE_BLK, 8), row_blk),
    ] + [pl.BlockSpec(w.shape, full) for w in ws]
    return pl.pallas_call(
        _score_body,
        grid=(grid,),
        in_specs=in_specs,
        out_specs=pl.BlockSpec((_SCORE_BLK, 8), row_blk),
        out_shape=jax.ShapeDtypeStruct((T_EDGES_C, 8), jnp.float32),
    )(tgi, tgj, feats8, *ws)


def _finalize_body(emb_ref, sgi_ref, sgj_ref, timp_ref,
                   gW1_ref, gb1_ref, gg1_ref, gB1_ref,
                   gW2_ref, gb2_ref, gg2_ref, gB2_ref,
                   mW_ref, mb_ref,
                   rW1_ref, rb1_ref, rg_ref, rB_ref,
                   rW2_ref, rb2_ref,
                   refined_ref, gemb_ref):
    emb = emb_ref[...]
    sgi = sgi_ref[...]
    sgj = sgj_ref[...]
    iota = lax.broadcasted_iota(jnp.int32, (TOPK_C, N_EDGES_C), 1)
    oh_i = (sgi == iota).astype(jnp.float32)
    oh_j = (sgj == iota).astype(jnp.float32)
    ei = _dot(oh_i, emb)
    ej = _dot(oh_j, emb)
    cc = jnp.concatenate([ei, ej], axis=-1)
    h = cc @ gW1_ref[...].T + gb1_ref[...]
    h = 0.5 * h * (1.0 + lax.erf(h / np.sqrt(2.0).astype(np.float32)))
    h = _layernorm(h, gg1_ref[...], gB1_ref[...])
    h = jnp.maximum(h @ gW2_ref[...].T + gb2_ref[...], 0.0)
    h = _layernorm(h, gg2_ref[...], gB2_ref[...])
    gwi = jnp.concatenate([h, timp_ref[...]], axis=-1)
    gemb = jnp.maximum(gwi @ mW_ref[...].T + mb_ref[...], 0.0)
    gemb_ref[...] = gemb
    ohsum = oh_i + oh_j
    ctx = _dot_t0(ohsum, gemb)
    cnt = jnp.sum(ohsum, axis=0)[:, None]
    ctx = jnp.where(cnt > 0, ctx / jnp.maximum(cnt, 1.0), ctx)
    comb = jnp.concatenate([emb, ctx], axis=-1)
    r = jnp.maximum(comb @ rW1_ref[...].T + rb1_ref[...], 0.0)
    r = _layernorm(r, rg_ref[...], rB_ref[...])
    refined_ref[...] = r @ rW2_ref[...].T + rb2_ref[...]


@jax.jit
def _finalize(emb, sgi, sgj, top_imp, p):
    out_shapes = (
        jax.ShapeDtypeStruct((N_EDGES_C, EDGE_DIM_C), jnp.float32),
        jax.ShapeDtypeStruct((TOPK_C, HIDDEN_C), jnp.float32),
    )
    args = (
        emb, sgi[:, None], sgj[:, None], top_imp[:, None],
        p['gW1'], p['gb1'][None, :], p['gg1'][None, :], p['gB1'][None, :],
        p['gW2'], p['gb2'][None, :], p['gg2'][None, :], p['gB2'][None, :],
        p['mW'], p['mb'][None, :],
        p['rW1'], p['rb1'][None, :], p['rg'][None, :], p['rB'][None, :],
        p['rW2'], p['rb2'][None, :],
    )
    return pl.pallas_call(
        _finalize_body,
        out_shape=out_shapes,
    )(*args)


def kernel(edge_embeddings, original_edge_index, transformed_edge_index, params):
    p = params
    oe = original_edge_index
    src, dst = transformed_edge_index[0], transformed_edge_index[1]
    m = src < dst
    pid = jnp.where(m, src * N_EDGES_C + dst, SENT_C)
    spid = jnp.sort(pid)
    prev = jnp.concatenate([jnp.full((1,), -1, spid.dtype), spid[:-1]])
    valid = (spid < SENT_C) & (spid != prev)
    gi = (spid // N_EDGES_C).astype(jnp.int32)
    gj = (spid % N_EDGES_C).astype(jnp.int32)

    x = edge_embeddings
    qkv = x @ p['Wqkv'].T + p['bqkv']
    q, k, v = jnp.split(qkv, 3, axis=-1)
    dh = EDGE_DIM_C // HEADS_C

    def sp(t):
        return t.reshape(-1, HEADS_C, dh).transpose(1, 0, 2)
    q, k, v = sp(q), sp(k), sp(v)
    attn = jax.nn.softmax(q @ k.transpose(0, 2, 1) / np.sqrt(dh), axis=-1)
    o = (attn @ v).transpose(1, 0, 2).reshape(-1, EDGE_DIM_C)
    emb = o @ p['Wo'].T + p['bo']
    deg = jnp.bincount(jnp.concatenate([oe[0], oe[1]]), length=N_NODES_C).astype(jnp.float32)
    d0 = deg[oe[0]]
    d1 = deg[oe[1]]
    gic = jnp.minimum(gi, N_EDGES_C - 1)
    gjc = jnp.minimum(gj, N_EDGES_C - 1)
    tgi, tgj = _gather(emb, gic, gjc)
    d2 = jnp.stack([d0, d1], axis=1)
    feats8 = jnp.concatenate([d2[gic], d2[gjc],
                              jnp.zeros((T_EDGES_C, 4), jnp.float32)], axis=1)
    z = _score(tgi, tgj, feats8, p)
    imp = jax.nn.sigmoid(z[:, 0])
    imp = jnp.where(valid, imp, -jnp.inf)
    top_imp, top_idx = jax.lax.top_k(imp, TOPK_C)
    sgi = gi[top_idx]
    sgj = gj[top_idx]

    refined, gemb = _finalize(emb, sgi, sgj, top_imp, p)
    return refined, gemb, top_imp, jnp.stack([sgi, sgj], axis=1)


# speedup vs baseline: 2.9035x; 2.9035x over previous
import functools

import jax
import jax.numpy as jnp
import numpy as np
from jax import lax
from jax.experimental import pallas as pl
from jax.experimental.pallas import tpu as pltpu

N_EDGES_C = 2048
EDGE_DIM_C = 128
HIDDEN_C = 256
HEADS_C = 4
TOPK_C = 512
N_NODES_C = 1024
T_EDGES_C = 32768
SENT_C = N_EDGES_C * N_EDGES_C


def _dot(a, b):
    return lax.dot_general(a, b, (((1,), (0,)), ((), ())),
                           preferred_element_type=jnp.float32,
                           precision=lax.Precision.HIGHEST)


def _dot_t0(a, b):
    return lax.dot_general(a, b, (((0,), (0,)), ((), ())),
                           preferred_element_type=jnp.float32,
                           precision=lax.Precision.HIGHEST)


def _layernorm(x, g, b, eps=1e-5):
    mu = jnp.mean(x, -1, keepdims=True)
    var = jnp.mean((x - mu) ** 2, -1, keepdims=True)
    return (x - mu) / jnp.sqrt(var + eps) * g + b


def _score_body(tgi_ref, tgj_ref, feats_ref,
                cW1_ref, cb1_ref, cW2_ref, cb2_ref,
                sW1_ref, sb1_ref, sW2_ref, sb2_ref,
                kW1_ref, kb1_ref, kW2_ref, kb2_ref,
                z_ref):
    dt = functools.partial(lax.dot_general, dimension_numbers=(((1,), (1,)), ((), ())),
                           preferred_element_type=jnp.float32)
    pair = jnp.concatenate([tgi_ref[...], tgj_ref[...]], axis=-1)
    c1 = jnp.maximum(dt(pair, cW1_ref[...]) + cb1_ref[...], 0.0)
    comp = jnp.maximum(dt(c1, cW2_ref[...]) + cb2_ref[...], 0.0)
    feats = feats_ref[...]
    s1 = jnp.maximum(dt(feats, sW1_ref[...]) + sb1_ref[...], 0.0)
    struct8 = dt(s1, sW2_ref[...]) + sb2_ref[...]
    allsc8 = jnp.concatenate([comp[:, :4], struct8[:, :4]], axis=-1)
    z1 = jnp.maximum(dt(allsc8, kW1_ref[...]) + kb1_ref[...], 0.0)
    z_ref[...] = dt(z1, kW2_ref[...]) + kb2_ref[...]


_SCORE_BLK = 4096


@jax.jit
def _score(tgi, tgj, feats8, p):
    z2 = lambda r, c: jnp.zeros((r, c), jnp.float32)
    cW2p = jnp.concatenate([p['cW2'], z2(4, HIDDEN_C)], 0)
    cb2p = jnp.concatenate([p['cb2'], jnp.zeros((4,), jnp.float32)])[None, :]
    sW1p = jnp.concatenate([p['sW1'], z2(64, 4)], 1)
    sW2p = jnp.concatenate([p['sW2'], z2(7, 64)], 0)
    sb2p = jnp.concatenate([p['sb2'], jnp.zeros((7,), jnp.float32)])[None, :]
    kW1p = jnp.concatenate([p['kW1'], z2(128, 3)], 1)
    kW2p = jnp.concatenate([p['kW2'], z2(7, 128)], 0)
    kb2p = jnp.concatenate([p['kb2'], jnp.zeros((7,), jnp.float32)])[None, :]
    ws = (p['cW1'], p['cb1'][None, :], cW2p, cb2p,
          sW1p, p['sb1'][None, :], sW2p, sb2p,
          kW1p, p['kb1'][None, :], kW2p, kb2p)
    grid = T_EDGES_C // _SCORE_BLK
    row_blk = lambda i: (i, 0)
    full = lambda i: (0, 0)
    in_specs = [
        pl.BlockSpec((_SCORE_BLK, tgi.shape[1]), row_blk),
        pl.BlockSpec((_SCORE_BLK, tgj.shape[1]), row_blk),
        pl.BlockSpec((_SCORE_BLK, 8), row_blk),
    ] + [pl.BlockSpec(w.shape, full) for w in ws]
    return pl.pallas_call(
        _score_body,
        grid=(grid,),
        in_specs=in_specs,
        out_specs=pl.BlockSpec((_SCORE_BLK, 8), row_blk),
        out_shape=jax.ShapeDtypeStruct((T_EDGES_C, 8), jnp.float32),
    )(tgi, tgj, feats8, *ws)


def _finalize_body(emb_ref, sgi_ref, sgj_ref, timp_ref,
                   gW1_ref, gb1_ref, gg1_ref, gB1_ref,
                   gW2_ref, gb2_ref, gg2_ref, gB2_ref,
                   mW_ref, mb_ref,
                   rW1_ref, rb1_ref, rg_ref, rB_ref,
                   rW2_ref, rb2_ref,
                   refined_ref, gemb_ref):
    emb = emb_ref[...]
    sgi = sgi_ref[...]
    sgj = sgj_ref[...]
    iota = lax.broadcasted_iota(jnp.int32, (TOPK_C, N_EDGES_C), 1)
    oh_i = (sgi == iota).astype(jnp.float32)
    oh_j = (sgj == iota).astype(jnp.float32)
    ei = _dot(oh_i, emb)
    ej = _dot(oh_j, emb)
    cc = jnp.concatenate([ei, ej], axis=-1)
    h = cc @ gW1_ref[...].T + gb1_ref[...]
    h = 0.5 * h * (1.0 + lax.erf(h / np.sqrt(2.0).astype(np.float32)))
    h = _layernorm(h, gg1_ref[...], gB1_ref[...])
    h = jnp.maximum(h @ gW2_ref[...].T + gb2_ref[...], 0.0)
    h = _layernorm(h, gg2_ref[...], gB2_ref[...])
    gwi = jnp.concatenate([h, timp_ref[...]], axis=-1)
    gemb = jnp.maximum(gwi @ mW_ref[...].T + mb_ref[...], 0.0)
    gemb_ref[...] = gemb
    ohsum = oh_i + oh_j
    ctx = _dot_t0(ohsum, gemb)
    cnt = jnp.sum(ohsum, axis=0)[:, None]
    ctx = jnp.where(cnt > 0, ctx / jnp.maximum(cnt, 1.0), ctx)
    comb = jnp.concatenate([emb, ctx], axis=-1)
    r = jnp.maximum(comb @ rW1_ref[...].T + rb1_ref[...], 0.0)
    r = _layernorm(r, rg_ref[...], rB_ref[...])
    refined_ref[...] = r @ rW2_ref[...].T + rb2_ref[...]


@jax.jit
def _finalize(emb, sgi, sgj, top_imp, p):
    out_shapes = (
        jax.ShapeDtypeStruct((N_EDGES_C, EDGE_DIM_C), jnp.float32),
        jax.ShapeDtypeStruct((TOPK_C, HIDDEN_C), jnp.float32),
    )
    args = (
        emb, sgi[:, None], sgj[:, None], top_imp[:, None],
        p['gW1'], p['gb1'][None, :], p['gg1'][None, :], p['gB1'][None, :],
        p['gW2'], p['gb2'][None, :], p['gg2'][None, :], p['gB2'][None, :],
        p['mW'], p['mb'][None, :],
        p['rW1'], p['rb1'][None, :], p['rg'][None, :], p['rB'][None, :],
        p['rW2'], p['rb2'][None, :],
    )
    return pl.pallas_call(
        _finalize_body,
        out_shape=out_shapes,
    )(*args)


def kernel(edge_embeddings, original_edge_index, transformed_edge_index, params):
    p = params
    oe = original_edge_index
    src, dst = transformed_edge_index[0], transformed_edge_index[1]
    m = src < dst
    pid = jnp.where(m, src * N_EDGES_C + dst, SENT_C)
    spid = jnp.sort(pid)
    prev = jnp.concatenate([jnp.full((1,), -1, spid.dtype), spid[:-1]])
    valid = (spid < SENT_C) & (spid != prev)
    gi = (spid // N_EDGES_C).astype(jnp.int32)
    gj = (spid % N_EDGES_C).astype(jnp.int32)

    x = edge_embeddings
    qkv = x @ p['Wqkv'].T + p['bqkv']
    q, k, v = jnp.split(qkv, 3, axis=-1)
    dh = EDGE_DIM_C // HEADS_C

    def sp(t):
        return t.reshape(-1, HEADS_C, dh).transpose(1, 0, 2)
    q, k, v = sp(q), sp(k), sp(v)
    attn = jax.nn.softmax(q @ k.transpose(0, 2, 1) / np.sqrt(dh), axis=-1)
    o = (attn @ v).transpose(1, 0, 2).reshape(-1, EDGE_DIM_C)
    emb = o @ p['Wo'].T + p['bo']
    deg = jnp.bincount(jnp.concatenate([oe[0], oe[1]]), length=N_NODES_C).astype(jnp.float32)
    d0 = deg[oe[0]]
    d1 = deg[oe[1]]
    tab = jnp.concatenate([emb, d0[:, None], d1[:, None],
                           jnp.zeros((N_EDGES_C, 2), jnp.float32)], axis=1)
    tg_i = tab[gi]
    tg_j = tab[gj]
    tgi = tg_i[:, :EDGE_DIM_C]
    tgj = tg_j[:, :EDGE_DIM_C]
    feats8 = jnp.concatenate(
        [tg_i[:, EDGE_DIM_C:EDGE_DIM_C + 2], tg_j[:, EDGE_DIM_C:EDGE_DIM_C + 2],
         jnp.zeros((T_EDGES_C, 4), jnp.float32)], axis=1)
    z = _score(tgi, tgj, feats8, p)
    imp = jax.nn.sigmoid(z[:, 0])
    imp = jnp.where(valid, imp, -jnp.inf)
    top_imp, top_idx = jax.lax.top_k(imp, TOPK_C)
    sgi = gi[top_idx]
    sgj = gj[top_idx]

    refined, gemb = _finalize(emb, sgi, sgj, top_imp, p)
    return refined, gemb, top_imp, jnp.stack([sgi, sgj], axis=1)


# speedup vs baseline: 3.2664x; 1.1250x over previous
import functools

import jax
import jax.numpy as jnp
import numpy as np
from jax import lax
from jax.experimental import pallas as pl
from jax.experimental.pallas import tpu as pltpu

N_EDGES_C = 2048
EDGE_DIM_C = 128
HIDDEN_C = 256
HEADS_C = 4
TOPK_C = 512
N_NODES_C = 1024
T_EDGES_C = 32768
SENT_C = N_EDGES_C * N_EDGES_C


def _dot(a, b):
    return lax.dot_general(a, b, (((1,), (0,)), ((), ())),
                           preferred_element_type=jnp.float32,
                           precision=lax.Precision.HIGHEST)


def _dot_t0(a, b):
    return lax.dot_general(a, b, (((0,), (0,)), ((), ())),
                           preferred_element_type=jnp.float32,
                           precision=lax.Precision.HIGHEST)


def _layernorm(x, g, b, eps=1e-5):
    mu = jnp.mean(x, -1, keepdims=True)
    var = jnp.mean((x - mu) ** 2, -1, keepdims=True)
    return (x - mu) / jnp.sqrt(var + eps) * g + b


def _score_body(tgi_ref, tgj_ref,
                cW1_ref, cb1_ref, cW2_ref, cb2_ref,
                sW1_ref, sb1_ref, sW2_ref, sb2_ref,
                kW1_ref, kb1_ref, kW2_ref, kb2_ref,
                z_ref):
    dt = functools.partial(lax.dot_general, dimension_numbers=(((1,), (1,)), ((), ())),
                           preferred_element_type=jnp.float32)
    tgi = tgi_ref[...]
    tgj = tgj_ref[...]
    pair = jnp.concatenate([tgi[:, :EDGE_DIM_C], tgj[:, :EDGE_DIM_C]], axis=-1)
    c1 = jnp.maximum(dt(pair, cW1_ref[...]) + cb1_ref[...], 0.0)
    comp = jnp.maximum(dt(c1, cW2_ref[...]) + cb2_ref[...], 0.0)
    feats = jnp.concatenate(
        [tgi[:, EDGE_DIM_C:EDGE_DIM_C + 4], tgj[:, EDGE_DIM_C:EDGE_DIM_C + 4]], axis=-1)
    s1 = jnp.maximum(dt(feats, sW1_ref[...]) + sb1_ref[...], 0.0)
    struct8 = dt(s1, sW2_ref[...]) + sb2_ref[...]
    allsc8 = jnp.concatenate([comp[:, :4], struct8[:, :4]], axis=-1)
    z1 = jnp.maximum(dt(allsc8, kW1_ref[...]) + kb1_ref[...], 0.0)
    z_ref[...] = dt(z1, kW2_ref[...]) + kb2_ref[...]


_SCORE_BLK = 4096


@jax.jit
def _score(tgi, tgj, p):
    z2 = lambda r, c: jnp.zeros((r, c), jnp.float32)
    cW2p = jnp.concatenate([p['cW2'], z2(4, HIDDEN_C)], 0)
    cb2p = jnp.concatenate([p['cb2'], jnp.zeros((4,), jnp.float32)])[None, :]
    sW1p = jnp.concatenate([p['sW1'][:, :2], z2(64, 2),
                            p['sW1'][:, 2:], z2(64, 2)], 1)
    sW2p = jnp.concatenate([p['sW2'], z2(7, 64)], 0)
    sb2p = jnp.concatenate([p['sb2'], jnp.zeros((7,), jnp.float32)])[None, :]
    kW1p = jnp.concatenate([p['kW1'], z2(128, 3)], 1)
    kW2p = jnp.concatenate([p['kW2'], z2(7, 128)], 0)
    kb2p = jnp.concatenate([p['kb2'], jnp.zeros((7,), jnp.float32)])[None, :]
    ws = (p['cW1'], p['cb1'][None, :], cW2p, cb2p,
          sW1p, p['sb1'][None, :], sW2p, sb2p,
          kW1p, p['kb1'][None, :], kW2p, kb2p)
    grid = T_EDGES_C // _SCORE_BLK
    row_blk = lambda i: (i, 0)
    full = lambda i: (0, 0)
    in_specs = [
        pl.BlockSpec((_SCORE_BLK, tgi.shape[1]), row_blk),
        pl.BlockSpec((_SCORE_BLK, tgj.shape[1]), row_blk),
    ] + [pl.BlockSpec(w.shape, full) for w in ws]
    return pl.pallas_call(
        _score_body,
        grid=(grid,),
        in_specs=in_specs,
        out_specs=pl.BlockSpec((_SCORE_BLK, 8), row_blk),
        out_shape=jax.ShapeDtypeStruct((T_EDGES_C, 8), jnp.float32),
    )(tgi, tgj, *ws)


def _finalize_body(emb_ref, sgi_ref, sgj_ref, timp_ref,
                   gW1_ref, gb1_ref, gg1_ref, gB1_ref,
                   gW2_ref, gb2_ref, gg2_ref, gB2_ref,
                   mW_ref, mb_ref,
                   rW1_ref, rb1_ref, rg_ref, rB_ref,
                   rW2_ref, rb2_ref,
                   refined_ref, gemb_ref):
    emb = emb_ref[...]
    sgi = sgi_ref[...]
    sgj = sgj_ref[...]
    iota = lax.broadcasted_iota(jnp.int32, (TOPK_C, N_EDGES_C), 1)
    oh_i = (sgi == iota).astype(jnp.float32)
    oh_j = (sgj == iota).astype(jnp.float32)
    ei = _dot(oh_i, emb)
    ej = _dot(oh_j, emb)
    cc = jnp.concatenate([ei, ej], axis=-1)
    h = cc @ gW1_ref[...].T + gb1_ref[...]
    h = 0.5 * h * (1.0 + lax.erf(h / np.sqrt(2.0).astype(np.float32)))
    h = _layernorm(h, gg1_ref[...], gB1_ref[...])
    h = jnp.maximum(h @ gW2_ref[...].T + gb2_ref[...], 0.0)
    h = _layernorm(h, gg2_ref[...], gB2_ref[...])
    gwi = jnp.concatenate([h, timp_ref[...]], axis=-1)
    gemb = jnp.maximum(gwi @ mW_ref[...].T + mb_ref[...], 0.0)
    gemb_ref[...] = gemb
    ohsum = oh_i + oh_j
    ctx = _dot_t0(ohsum, gemb)
    cnt = jnp.sum(ohsum, axis=0)[:, None]
    ctx = jnp.where(cnt > 0, ctx / jnp.maximum(cnt, 1.0), ctx)
    comb = jnp.concatenate([emb, ctx], axis=-1)
    r = jnp.maximum(comb @ rW1_ref[...].T + rb1_ref[...], 0.0)
    r = _layernorm(r, rg_ref[...], rB_ref[...])
    refined_ref[...] = r @ rW2_ref[...].T + rb2_ref[...]


@jax.jit
def _finalize(emb, sgi, sgj, top_imp, p):
    out_shapes = (
        jax.ShapeDtypeStruct((N_EDGES_C, EDGE_DIM_C), jnp.float32),
        jax.ShapeDtypeStruct((TOPK_C, HIDDEN_C), jnp.float32),
    )
    args = (
        emb, sgi[:, None], sgj[:, None], top_imp[:, None],
        p['gW1'], p['gb1'][None, :], p['gg1'][None, :], p['gB1'][None, :],
        p['gW2'], p['gb2'][None, :], p['gg2'][None, :], p['gB2'][None, :],
        p['mW'], p['mb'][None, :],
        p['rW1'], p['rb1'][None, :], p['rg'][None, :], p['rB'][None, :],
        p['rW2'], p['rb2'][None, :],
    )
    return pl.pallas_call(
        _finalize_body,
        out_shape=out_shapes,
    )(*args)


def kernel(edge_embeddings, original_edge_index, transformed_edge_index, params):
    p = params
    oe = original_edge_index
    src, dst = transformed_edge_index[0], transformed_edge_index[1]
    m = src < dst
    pid = jnp.where(m, src * N_EDGES_C + dst, SENT_C)
    spid = jnp.sort(pid)
    prev = jnp.concatenate([jnp.full((1,), -1, spid.dtype), spid[:-1]])
    valid = (spid < SENT_C) & (spid != prev)
    gi = (spid // N_EDGES_C).astype(jnp.int32)
    gj = (spid % N_EDGES_C).astype(jnp.int32)

    x = edge_embeddings
    qkv = x @ p['Wqkv'].T + p['bqkv']
    q, k, v = jnp.split(qkv, 3, axis=-1)
    dh = EDGE_DIM_C // HEADS_C

    def sp(t):
        return t.reshape(-1, HEADS_C, dh).transpose(1, 0, 2)
    q, k, v = sp(q), sp(k), sp(v)
    attn = jax.nn.softmax(q @ k.transpose(0, 2, 1) / np.sqrt(dh), axis=-1)
    o = (attn @ v).transpose(1, 0, 2).reshape(-1, EDGE_DIM_C)
    emb = o @ p['Wo'].T + p['bo']
    deg = jnp.bincount(jnp.concatenate([oe[0], oe[1]]), length=N_NODES_C).astype(jnp.float32)
    d0 = deg[oe[0]]
    d1 = deg[oe[1]]
    tab = jnp.concatenate([emb, d0[:, None], d1[:, None],
                           jnp.zeros((N_EDGES_C, 2), jnp.float32)], axis=1)
    tg_i = tab[gi]
    tg_j = tab[gj]
    z = _score(tg_i, tg_j, p)
    imp = jax.nn.sigmoid(z[:, 0])
    imp = jnp.where(valid, imp, -jnp.inf)
    top_imp, top_idx = jax.lax.top_k(imp, TOPK_C)
    sgi = gi[top_idx]
    sgj = gj[top_idx]

    refined, gemb = _finalize(emb, sgi, sgj, top_imp, p)
    return refined, gemb, top_imp, jnp.stack([sgi, sgj], axis=1)
